# baseline (speedup 1.0000x reference)
import functools

import jax
import jax.numpy as jnp
from jax import lax
from jax.experimental import pallas as pl
from jax.experimental.pallas import tpu as pltpu
from jax.experimental.pallas import tpu_sc as plsc

_ROWS = 4096
_COLS = 200
_D = 64
_B = _ROWS * _COLS
_NC = 2
_NS = 16
_NW = _NC * _NS
_BPW = _B // _NW
_C = 1600
_NCHUNK = _BPW // _C


def _gather_body(x_hbm, W_hbm, out_hbm, idx_v, rows_v, sem):
    wid = lax.axis_index("s") * _NC + lax.axis_index("c")
    base = wid * _BPW

    def step(i, carry):
        off = base + i * _C
        pltpu.sync_copy(x_hbm.at[pl.ds(off, _C)], idx_v)
        pltpu.async_copy(W_hbm.at[idx_v], rows_v, sem).wait()
        pltpu.sync_copy(rows_v, out_hbm.at[pl.ds(off, _C)])
        return carry

    lax.fori_loop(0, _NCHUNK, step, 0)


_gather = functools.partial(
    pl.kernel,
    out_type=jax.ShapeDtypeStruct((_B, _D), jnp.float32),
    mesh=plsc.VectorSubcoreMesh(core_axis_name="c", subcore_axis_name="s"),
    scratch_types=[
        pltpu.VMEM((_C,), jnp.int32),
        pltpu.VMEM((_C, _D), jnp.float32),
        pltpu.SemaphoreType.DMA,
    ],
    compiler_params=pltpu.CompilerParams(use_tc_tiling_on_sc=False),
)(_gather_body)


def _mask_body(x_ref, m_ref):
    m_ref[...] = x_ref[...] != 0


_mask = pl.pallas_call(
    _mask_body,
    out_shape=jax.ShapeDtypeStruct((_ROWS, _COLS), jnp.bool_),
)


def kernel(x, W):
    out = _gather(x.reshape(_B), W)
    pad_mask = _mask(x)
    return out.reshape(_ROWS, _COLS, _D), pad_mask
